# Initial kernel scaffold; baseline (speedup 1.0000x reference)
import jax
import jax.numpy as jnp
from jax.experimental import pallas as pl


def kernel(gains, bws, pms, gain_table, bw_table, pm_table, W_proj, b_proj, W_fc, b_fc):
    raise NotImplementedError("write your pallas kernel here")


# speedup vs baseline: 3.1623x; 3.1623x over previous
import functools

import jax
import jax.numpy as jnp
from jax import lax
from jax.experimental import pallas as pl
from jax.experimental.pallas import tpu as pltpu
from jax.experimental.pallas import tpu_sc as plsc

B = 16384
EMB = 128
LAT = 64
CHUNK = 128

_NC, _NS = 2, 16
_NW = _NC * _NS
_BPW = B // _NW
_NCHUNK = _BPW // CHUNK


@functools.cache
def _make_gather3():
    mesh = plsc.VectorSubcoreMesh(
        core_axis_name="c", subcore_axis_name="s", num_cores=_NC
    )

    @functools.partial(
        pl.kernel,
        mesh=mesh,
        out_type=jax.ShapeDtypeStruct((3 * B, EMB), jnp.float32),
        scratch_types=[
            pltpu.VMEM((_NCHUNK, CHUNK), jnp.int32),
            pltpu.VMEM((_BPW, EMB), jnp.float32),
            pltpu.SemaphoreType.DMA,
        ],
    )
    def gather3(g_hbm, b_hbm, p_hbm, gt_hbm, bt_hbm, pt_hbm, out_hbm, idx_v, rows_v, sem):
        wid = lax.axis_index("s") * _NC + lax.axis_index("c")
        base = wid * _BPW
        for t, (idx_hbm, tab_hbm) in enumerate(
            ((g_hbm, gt_hbm), (b_hbm, bt_hbm), (p_hbm, pt_hbm))
        ):
            for j in range(_NCHUNK):
                pltpu.sync_copy(idx_hbm.at[pl.ds(base + j * CHUNK, CHUNK)], idx_v.at[j])
            copies = [
                pltpu.async_copy(
                    tab_hbm.at[idx_v.at[j]], rows_v.at[pl.ds(j * CHUNK, CHUNK)], sem
                )
                for j in range(_NCHUNK)
            ]
            for c in copies:
                c.wait()
            pltpu.sync_copy(rows_v, out_hbm.at[pl.ds(t * B + base, _BPW)])

    return gather3


BLK = 1024


def _proj_body(xg_ref, xb_ref, xp_ref, wp_ref, bp_ref, wf_ref, bf_ref, o_ref):
    h = jnp.dot(xg_ref[...], wp_ref[0:EMB, :], preferred_element_type=jnp.float32)
    h = h + jnp.dot(xb_ref[...], wp_ref[EMB : 2 * EMB, :], preferred_element_type=jnp.float32)
    h = h + jnp.dot(xp_ref[...], wp_ref[2 * EMB : 3 * EMB, :], preferred_element_type=jnp.float32)
    h = h + bp_ref[...]
    o_ref[...] = jnp.dot(h, wf_ref[...], preferred_element_type=jnp.float32) + bf_ref[...]


def _proj(gathered, W_proj, b_proj, W_fc, b_fc):
    nblk = B // BLK
    return pl.pallas_call(
        _proj_body,
        grid=(nblk,),
        in_specs=[
            pl.BlockSpec((BLK, EMB), lambda i: (i, 0)),
            pl.BlockSpec((BLK, EMB), lambda i: (i + B // BLK, 0)),
            pl.BlockSpec((BLK, EMB), lambda i: (i + 2 * (B // BLK), 0)),
            pl.BlockSpec((3 * EMB, EMB), lambda i: (0, 0)),
            pl.BlockSpec((1, EMB), lambda i: (0, 0)),
            pl.BlockSpec((EMB, LAT), lambda i: (0, 0)),
            pl.BlockSpec((1, LAT), lambda i: (0, 0)),
        ],
        out_specs=pl.BlockSpec((BLK, LAT), lambda i: (i, 0)),
        out_shape=jax.ShapeDtypeStruct((B, LAT), jnp.float32),
    )(gathered, gathered, gathered, W_proj, b_proj.reshape(1, EMB), W_fc, b_fc.reshape(1, LAT))


def kernel(gains, bws, pms, gain_table, bw_table, pm_table, W_proj, b_proj, W_fc, b_fc):
    g = gains.astype(jnp.int32)
    bw = bws.astype(jnp.int32)
    pm = pms.astype(jnp.int32)
    gathered = _make_gather3()(g, bw, pm, gain_table, bw_table, pm_table)
    return _proj(gathered, W_proj, b_proj, W_fc, b_fc)


# speedup vs baseline: 3.6828x; 1.1646x over previous
import functools

import jax
import jax.numpy as jnp
from jax import lax
from jax.experimental import pallas as pl
from jax.experimental.pallas import tpu as pltpu
from jax.experimental.pallas import tpu_sc as plsc

B = 16384
EMB = 128
LAT = 64
CHUNK = 128
STAGE = 256
NBUF = 3

_NC, _NS = 2, 16
_NW = _NC * _NS
_BPW = B // _NW
_NSTAGE = 3 * (_BPW // STAGE)


@functools.cache
def _make_gather3():
    mesh = plsc.VectorSubcoreMesh(
        core_axis_name="c", subcore_axis_name="s", num_cores=_NC
    )

    @functools.partial(
        pl.kernel,
        mesh=mesh,
        out_type=(
            jax.ShapeDtypeStruct((B, EMB), jnp.float32),
            jax.ShapeDtypeStruct((B, EMB), jnp.float32),
            jax.ShapeDtypeStruct((B, EMB), jnp.float32),
        ),
    scratch_types=[
            pltpu.VMEM((_BPW,), jnp.int32),
            pltpu.VMEM((_BPW,), jnp.int32),
            pltpu.VMEM((_BPW,), jnp.int32),
            pltpu.VMEM((STAGE, EMB), jnp.float32),
            pltpu.VMEM((STAGE, EMB), jnp.float32),
            pltpu.VMEM((STAGE, EMB), jnp.float32),
            pltpu.SemaphoreType.DMA,
            pltpu.SemaphoreType.DMA,
            pltpu.SemaphoreType.DMA,
        ],
    )
    def gather3(
        g_hbm, b_hbm, p_hbm, gt_hbm, bt_hbm, pt_hbm,
        og_hbm, ob_hbm, op_hbm, ig_v, ib_v, ip_v, r0_v, r1_v, r2_v,
        isem, gsem, wsem,
    ):
        wid = lax.axis_index("s") * _NC + lax.axis_index("c")
        base = wid * _BPW
        tabs = (gt_hbm, bt_hbm, pt_hbm)
        outs = (og_hbm, ob_hbm, op_hbm)
        idxs = (ig_v, ib_v, ip_v)
        bufs = (r0_v, r1_v, r2_v)
        icopies = [
            pltpu.async_copy(idx.at[pl.ds(base, _BPW)], idxs[t], isem)
            for t, idx in enumerate((g_hbm, b_hbm, p_hbm))
        ]
        for c in icopies:
            c.wait()

        spt = _BPW // STAGE

        def fire_gather(s):
            t, h = s // spt, s % spt
            return [
                pltpu.async_copy(
                    tabs[t].at[idxs[t].at[pl.ds(h * STAGE + j * CHUNK, CHUNK)]],
                    bufs[s % NBUF].at[pl.ds(j * CHUNK, CHUNK)],
                    gsem,
                )
                for j in range(STAGE // CHUNK)
            ]

        def fire_write(s):
            t, h = s // spt, s % spt
            return pltpu.async_copy(
                bufs[s % NBUF],
                outs[t].at[pl.ds(base + h * STAGE, STAGE)],
                wsem,
            )

        gathers = {0: fire_gather(0)}
        writes = {}
        for s in range(_NSTAGE):
            if s + 1 < _NSTAGE:
                if s + 1 >= NBUF:
                    writes[s + 1 - NBUF].wait()
                gathers[s + 1] = fire_gather(s + 1)
            for c in gathers[s]:
                c.wait()
            writes[s] = fire_write(s)
        for s in range(_NSTAGE - NBUF, _NSTAGE):
            writes[s].wait()

    return gather3


BLK = 2048


def _proj_body(xg_ref, xb_ref, xp_ref, wp_ref, bp_ref, wf_ref, bf_ref, o_ref):
    h = jnp.dot(xg_ref[...], wp_ref[0:EMB, :], preferred_element_type=jnp.float32)
    h = h + jnp.dot(xb_ref[...], wp_ref[EMB : 2 * EMB, :], preferred_element_type=jnp.float32)
    h = h + jnp.dot(xp_ref[...], wp_ref[2 * EMB : 3 * EMB, :], preferred_element_type=jnp.float32)
    h = h + bp_ref[...]
    o_ref[...] = jnp.dot(h, wf_ref[...], preferred_element_type=jnp.float32) + bf_ref[...]


def _proj(xg, xb, xp, W_proj, b_proj, W_fc, b_fc):
    nblk = B // BLK
    return pl.pallas_call(
        _proj_body,
        grid=(nblk,),
        in_specs=[
            pl.BlockSpec((BLK, EMB), lambda i: (i, 0)),
            pl.BlockSpec((BLK, EMB), lambda i: (i, 0)),
            pl.BlockSpec((BLK, EMB), lambda i: (i, 0)),
            pl.BlockSpec((3 * EMB, EMB), lambda i: (0, 0)),
            pl.BlockSpec((1, EMB), lambda i: (0, 0)),
            pl.BlockSpec((EMB, LAT), lambda i: (0, 0)),
            pl.BlockSpec((1, LAT), lambda i: (0, 0)),
        ],
        out_specs=pl.BlockSpec((BLK, LAT), lambda i: (i, 0)),
        out_shape=jax.ShapeDtypeStruct((B, LAT), jnp.float32),
    )(xg, xb, xp, W_proj, b_proj.reshape(1, EMB), W_fc, b_fc.reshape(1, LAT))


def kernel(gains, bws, pms, gain_table, bw_table, pm_table, W_proj, b_proj, W_fc, b_fc):
    g = gains.astype(jnp.int32)
    bw = bws.astype(jnp.int32)
    pm = pms.astype(jnp.int32)
    xg, xb, xp = _make_gather3()(g, bw, pm, gain_table, bw_table, pm_table)
    return _proj(xg, xb, xp, W_proj, b_proj, W_fc, b_fc)


# speedup vs baseline: 4.3879x; 1.1915x over previous
import functools

import jax
import jax.numpy as jnp
from jax import lax
from jax.experimental import pallas as pl
from jax.experimental.pallas import tpu as pltpu
from jax.experimental.pallas import tpu_sc as plsc

B = 16384
EMB = 128
LAT = 64
CHUNK = 128
STAGE = 256
NBUF = 3

_NC, _NS = 2, 16
_NW = _NC * _NS
_BPW = B // _NW
_NSTAGE = 3 * (_BPW // STAGE)


@functools.cache
def _make_gather3():
    mesh = plsc.VectorSubcoreMesh(
        core_axis_name="c", subcore_axis_name="s", num_cores=_NC
    )

    @functools.partial(
        pl.kernel,
        mesh=mesh,
        out_type=(
            jax.ShapeDtypeStruct((B, EMB), jnp.float32),
            jax.ShapeDtypeStruct((B, EMB), jnp.float32),
            jax.ShapeDtypeStruct((B, EMB), jnp.float32),
        ),
    scratch_types=[
            pltpu.VMEM((_BPW,), jnp.int32),
            pltpu.VMEM((_BPW,), jnp.int32),
            pltpu.VMEM((_BPW,), jnp.int32),
            pltpu.VMEM((STAGE, EMB), jnp.float32),
            pltpu.VMEM((STAGE, EMB), jnp.float32),
            pltpu.VMEM((STAGE, EMB), jnp.float32),
            pltpu.SemaphoreType.DMA,
            pltpu.SemaphoreType.DMA,
            pltpu.SemaphoreType.DMA,
        ],
    )
    def gather3(
        g_hbm, b_hbm, p_hbm, gt_hbm, bt_hbm, pt_hbm,
        og_hbm, ob_hbm, op_hbm, ig_v, ib_v, ip_v, r0_v, r1_v, r2_v,
        isem, gsem, wsem,
    ):
        wid = lax.axis_index("s") * _NC + lax.axis_index("c")
        base = wid * _BPW
        tabs = (gt_hbm, bt_hbm, pt_hbm)
        outs = (og_hbm, ob_hbm, op_hbm)
        idxs = (ig_v, ib_v, ip_v)
        bufs = (r0_v, r1_v, r2_v)
        icopies = [
            pltpu.async_copy(idx.at[pl.ds(base, _BPW)], idxs[t], isem)
            for t, idx in enumerate((g_hbm, b_hbm, p_hbm))
        ]
        for c in icopies:
            c.wait()

        spt = _BPW // STAGE

        def fire_gather(s):
            t, h = s // spt, s % spt
            return [
                pltpu.async_copy(
                    tabs[t].at[idxs[t].at[pl.ds(h * STAGE + j * CHUNK, CHUNK)]],
                    bufs[s % NBUF].at[pl.ds(j * CHUNK, CHUNK)],
                    gsem,
                )
                for j in range(STAGE // CHUNK)
            ]

        def fire_write(s):
            t, h = s // spt, s % spt
            return pltpu.async_copy(
                bufs[s % NBUF],
                outs[t].at[pl.ds(base + h * STAGE, STAGE)],
                wsem,
            )

        gathers = {0: fire_gather(0)}
        writes = {}
        for s in range(_NSTAGE):
            if s + 1 < _NSTAGE:
                if s + 1 >= NBUF:
                    writes[s + 1 - NBUF].wait()
                gathers[s + 1] = fire_gather(s + 1)
            for c in gathers[s]:
                c.wait()
            writes[s] = fire_write(s)
        for s in range(_NSTAGE - NBUF, _NSTAGE):
            writes[s].wait()

    return gather3


BLK = 4096


def _proj_body(xg_ref, xb_ref, xp_ref, wp_ref, bp_ref, wf_ref, bf_ref, o_ref):
    h = jnp.dot(xg_ref[...], wp_ref[0:EMB, :], preferred_element_type=jnp.float32)
    h = h + jnp.dot(xb_ref[...], wp_ref[EMB : 2 * EMB, :], preferred_element_type=jnp.float32)
    h = h + jnp.dot(xp_ref[...], wp_ref[2 * EMB : 3 * EMB, :], preferred_element_type=jnp.float32)
    h = h + bp_ref[...]
    ot = lax.dot_general(
        wf_ref[...], h, (((0,), (1,)), ((), ())),
        preferred_element_type=jnp.float32,
    )
    o_ref[...] = ot + bf_ref[...]


def _proj(xg, xb, xp, W_proj, b_proj, W_fc, b_fc):
    nblk = B // BLK
    outT = pl.pallas_call(
        _proj_body,
        grid=(nblk,),
        in_specs=[
            pl.BlockSpec((BLK, EMB), lambda i: (i, 0)),
            pl.BlockSpec((BLK, EMB), lambda i: (i, 0)),
            pl.BlockSpec((BLK, EMB), lambda i: (i, 0)),
            pl.BlockSpec((3 * EMB, EMB), lambda i: (0, 0)),
            pl.BlockSpec((1, EMB), lambda i: (0, 0)),
            pl.BlockSpec((EMB, LAT), lambda i: (0, 0)),
            pl.BlockSpec((LAT, 1), lambda i: (0, 0)),
        ],
        out_specs=pl.BlockSpec((LAT, BLK), lambda i: (0, i)),
        out_shape=jax.ShapeDtypeStruct((LAT, B), jnp.float32),
    )(xg, xb, xp, W_proj, b_proj.reshape(1, EMB), W_fc, b_fc.reshape(LAT, 1))
    return outT.T


def kernel(gains, bws, pms, gain_table, bw_table, pm_table, W_proj, b_proj, W_fc, b_fc):
    g = gains.astype(jnp.int32)
    bw = bws.astype(jnp.int32)
    pm = pms.astype(jnp.int32)
    xg, xb, xp = _make_gather3()(g, bw, pm, gain_table, bw_table, pm_table)
    return _proj(xg, xb, xp, W_proj, b_proj, W_fc, b_fc)
